# baseline (speedup 1.0000x reference)
import functools

import jax
import jax.numpy as jnp
from jax import lax
from jax.experimental import pallas as pl
from jax.experimental.pallas import tpu as pltpu
from jax.experimental.pallas import tpu_sc as plsc

CHUNK = 128
NBUF = 4
PADW = 128


def _emb_lookup(idx, tpad, D):
    B = idx.shape[0]
    NW = 32
    per_w = B // NW
    n_chunks = per_w // CHUNK
    assert n_chunks % NBUF == 0 and n_chunks // NBUF >= 2

    mesh = plsc.VectorSubcoreMesh(core_axis_name="c", subcore_axis_name="s")

    @functools.partial(
        pl.kernel,
        out_type=jax.ShapeDtypeStruct((B, D), jnp.float32),
        mesh=mesh,
        compiler_params=pltpu.CompilerParams(use_tc_tiling_on_sc=False),
        scratch_types=[
            pltpu.VMEM((per_w,), jnp.int32),
            [pltpu.VMEM((CHUNK, PADW), jnp.float32) for _ in range(NBUF)],
            [pltpu.SemaphoreType.DMA for _ in range(NBUF)],
        ],
    )
    def emb(idx_hbm, tpad_hbm, out_hbm, idx_v, bufs, sems):
        wid = lax.axis_index("s") * 2 + lax.axis_index("c")
        base = wid * per_w
        pltpu.sync_copy(idx_hbm.at[pl.ds(base, per_w)], idx_v)

        def gather(j, b):
            pltpu.async_copy(
                tpad_hbm.at[idx_v.at[pl.ds(j * CHUNK, CHUNK)]], bufs[b], sems[b]
            )

        def wait_gather(b):
            pltpu.make_async_copy(
                tpad_hbm.at[pl.ds(0, CHUNK)], bufs[b], sems[b]
            ).wait()

        for b in range(NBUF):
            gather(b, b)

        def outer(k, carry):
            j0 = k * NBUF
            for b in range(NBUF):
                j = j0 + b
                wait_gather(b)
                pltpu.sync_copy(
                    bufs[b].at[:, pl.ds(0, D)],
                    out_hbm.at[pl.ds(base + j * CHUNK, CHUNK)],
                )
                gather(j + NBUF, b)
            return carry

        lax.fori_loop(0, n_chunks // NBUF - 1, outer, 0, unroll=False)

        for b in range(NBUF):
            j = n_chunks - NBUF + b
            wait_gather(b)
            pltpu.sync_copy(
                bufs[b].at[:, pl.ds(0, D)],
                out_hbm.at[pl.ds(base + j * CHUNK, CHUNK)],
            )

    return emb(idx, tpad)


def kernel(x, table):
    B0, S = x.shape
    V, D = table.shape
    idx = x.reshape(-1).astype(jnp.int32)
    tpad = jnp.pad(table, ((0, 0), (0, 128 - D)))
    out = _emb_lookup(idx, tpad, D)
    return out.reshape(B0, S, D)
